# baseline (speedup 1.0000x reference)
import jax
import jax.numpy as jnp
import numpy as np
from jax.experimental import pallas as pl
from jax.experimental.pallas import tpu as pltpu

_EPS = 1e-06
_S = 0.025
_ALPHA = 0.98
_DELTA = 2.0
_R = 0.5

_TC = 256
_TBLK = 1024


def _pcen_kernel(x_ref, l_ref, o_ref, carry_ref):
    F = x_ref.shape[1]
    t = pl.program_id(1)

    @pl.when(t == 0)
    def _():
        carry_ref[...] = jnp.zeros_like(carry_ref)

    lmat = l_ref[...]
    decay = lmat[0:1, :] * ((1.0 - _S) / _S)
    lmat_bf = lmat.astype(jnp.bfloat16)
    sqrt_delta = np.float32(np.sqrt(_DELTA))

    carry = carry_ref[...]
    for c in range(_TBLK // _TC):
        x = x_ref[0, :, pl.ds(c * _TC, _TC)]
        y = jax.lax.dot_general(
            x.astype(jnp.bfloat16),
            lmat_bf,
            (((1,), (0,)), ((), ())),
            preferred_element_type=jnp.float32,
        )
        m = y + carry * decay
        carry = m[:, _TC - 1 : _TC]
        u = x * jnp.exp2(-_ALPHA * jnp.log2(m + _EPS)) + _DELTA
        o_ref[0, :, pl.ds(c * _TC, _TC)] = u * jax.lax.rsqrt(u) - sqrt_delta
    carry_ref[...] = carry


@jax.jit
def kernel(x):
    B, F, T = x.shape
    j = np.arange(_TC)
    diff = j[None, :] - j[:, None]
    L = np.where(diff >= 0, _S * (1.0 - _S) ** diff, 0.0)
    L = jnp.asarray(L, dtype=jnp.float32)

    return pl.pallas_call(
        _pcen_kernel,
        grid=(B, T // _TBLK),
        in_specs=[
            pl.BlockSpec((1, F, _TBLK), lambda b, t: (b, 0, t)),
            pl.BlockSpec((_TC, _TC), lambda b, t: (0, 0)),
        ],
        out_specs=pl.BlockSpec((1, F, _TBLK), lambda b, t: (b, 0, t)),
        out_shape=jax.ShapeDtypeStruct((B, F, T), jnp.float32),
        scratch_shapes=[pltpu.VMEM((F, 1), jnp.float32)],
        compiler_params=pltpu.CompilerParams(
            dimension_semantics=("parallel", "arbitrary")
        ),
    )(x, L)


# speedup vs baseline: 1.3284x; 1.3284x over previous
import jax
import jax.numpy as jnp
import numpy as np
from jax.experimental import pallas as pl
from jax.experimental.pallas import tpu as pltpu

_EPS = 1e-06
_S = 0.025
_ALPHA = 0.98
_DELTA = 2.0
_R = 0.5

_TC = 256
_TBLK = 2048


def _pcen_kernel(x_ref, l_ref, o_ref, carry_ref):
    F = x_ref.shape[1]
    t = pl.program_id(1)

    @pl.when(t == 0)
    def _():
        carry_ref[...] = jnp.zeros_like(carry_ref)

    lmat = l_ref[...]
    decay = lmat[0:1, :] * ((1.0 - _S) / _S)
    lmat_bf = lmat.astype(jnp.bfloat16)
    sqrt_delta = np.float32(np.sqrt(_DELTA))

    carry = carry_ref[...]
    for c in range(_TBLK // _TC):
        x = x_ref[0, :, pl.ds(c * _TC, _TC)]
        y = jax.lax.dot_general(
            x.astype(jnp.bfloat16),
            lmat_bf,
            (((1,), (0,)), ((), ())),
            preferred_element_type=jnp.float32,
        )
        m = y + carry * decay
        carry = m[:, _TC - 1 : _TC]
        u = x * jnp.exp2(-_ALPHA * jnp.log2(m + _EPS)) + _DELTA
        o_ref[0, :, pl.ds(c * _TC, _TC)] = u * jax.lax.rsqrt(u) - sqrt_delta
    carry_ref[...] = carry


@jax.jit
def kernel(x):
    B, F, T = x.shape
    j = np.arange(_TC)
    diff = j[None, :] - j[:, None]
    L = np.where(diff >= 0, _S * (1.0 - _S) ** diff, 0.0)
    L = jnp.asarray(L, dtype=jnp.float32)

    return pl.pallas_call(
        _pcen_kernel,
        grid=(B, T // _TBLK),
        in_specs=[
            pl.BlockSpec((1, F, _TBLK), lambda b, t: (b, 0, t)),
            pl.BlockSpec((_TC, _TC), lambda b, t: (0, 0)),
        ],
        out_specs=pl.BlockSpec((1, F, _TBLK), lambda b, t: (b, 0, t)),
        out_shape=jax.ShapeDtypeStruct((B, F, T), jnp.float32),
        scratch_shapes=[pltpu.VMEM((F, 1), jnp.float32)],
        compiler_params=pltpu.CompilerParams(
            dimension_semantics=("parallel", "arbitrary")
        ),
    )(x, L)


# speedup vs baseline: 1.6221x; 1.2211x over previous
import jax
import jax.numpy as jnp
import numpy as np
from jax.experimental import pallas as pl
from jax.experimental.pallas import tpu as pltpu

_EPS = 1e-06
_S = 0.025
_ALPHA = 0.98
_DELTA = 2.0
_R = 0.5

_TC = 256


def _pcen_kernel(x_ref, l_ref, o_hbm, obuf, ssem):
    F = x_ref.shape[1]
    T = x_ref.shape[2]
    b = pl.program_id(0)
    nb = pl.num_programs(0)
    slot = jax.lax.rem(b, 2)

    def store_copy(step):
        return pltpu.make_async_copy(
            obuf.at[jax.lax.rem(step, 2)], o_hbm.at[step], ssem.at[jax.lax.rem(step, 2)]
        )

    @pl.when(b >= 2)
    def _():
        store_copy(b - 2).wait()

    lmat = l_ref[...]
    decay = lmat[0:1, :] * ((1.0 - _S) / _S)
    lmat_bf = lmat.astype(jnp.bfloat16)
    sqrt_delta = np.float32(np.sqrt(_DELTA))

    carry = jnp.zeros((F, 1), jnp.float32)
    for c in range(T // _TC):
        x = x_ref[0, :, pl.ds(c * _TC, _TC)]
        y = jax.lax.dot_general(
            x.astype(jnp.bfloat16),
            lmat_bf,
            (((1,), (0,)), ((), ())),
            preferred_element_type=jnp.float32,
        )
        m = y + carry * decay
        carry = m[:, _TC - 1 : _TC]
        u = x * jnp.exp2(-_ALPHA * jnp.log2(m + _EPS)) + _DELTA
        obuf[slot, :, pl.ds(c * _TC, _TC)] = u * jax.lax.rsqrt(u) - sqrt_delta

    store_copy(b).start()

    @pl.when(b == nb - 1)
    def _():
        store_copy(b - 1).wait()
        store_copy(b).wait()


@jax.jit
def kernel(x):
    B, F, T = x.shape
    j = np.arange(_TC)
    diff = j[None, :] - j[:, None]
    L = np.where(diff >= 0, _S * (1.0 - _S) ** diff, 0.0)
    L = jnp.asarray(L, dtype=jnp.float32)

    return pl.pallas_call(
        _pcen_kernel,
        grid=(B,),
        in_specs=[
            pl.BlockSpec((1, F, T), lambda b: (b, 0, 0)),
            pl.BlockSpec((_TC, _TC), lambda b: (0, 0)),
        ],
        out_specs=pl.BlockSpec(memory_space=pltpu.MemorySpace.HBM),
        out_shape=jax.ShapeDtypeStruct((B, F, T), jnp.float32),
        scratch_shapes=[
            pltpu.VMEM((2, F, T), jnp.float32),
            pltpu.SemaphoreType.DMA((2,)),
        ],
        compiler_params=pltpu.CompilerParams(
            dimension_semantics=("arbitrary",)
        ),
    )(x, L)


# speedup vs baseline: 1.7669x; 1.0892x over previous
import jax
import jax.numpy as jnp
import numpy as np
from jax.experimental import pallas as pl
from jax.experimental.pallas import tpu as pltpu

_EPS = 1e-06
_S = 0.025
_ALPHA = 0.98
_DELTA = 2.0
_R = 0.5

_TC = 256
_BB = 2


def _pcen_kernel(x_ref, l_ref, o_ref):
    F = x_ref.shape[1]
    T = x_ref.shape[2]
    lmat = l_ref[...]
    decay = lmat[0:1, :] * ((1.0 - _S) / _S)
    lmat_bf = lmat.astype(jnp.bfloat16)
    sqrt_delta = np.float32(np.sqrt(_DELTA))

    for b in range(_BB):
        carry = jnp.zeros((F, 1), jnp.float32)
        for c in range(T // _TC):
            x = x_ref[b, :, pl.ds(c * _TC, _TC)]
            y = jax.lax.dot_general(
                x.astype(jnp.bfloat16),
                lmat_bf,
                (((1,), (0,)), ((), ())),
                preferred_element_type=jnp.float32,
            )
            m = y + carry * decay
            carry = m[:, _TC - 1 : _TC]
            u = x * jnp.exp2(-_ALPHA * jnp.log2(m + _EPS)) + _DELTA
            o_ref[b, :, pl.ds(c * _TC, _TC)] = u * jax.lax.rsqrt(u) - sqrt_delta


@jax.jit
def kernel(x):
    B, F, T = x.shape
    j = np.arange(_TC)
    diff = j[None, :] - j[:, None]
    L = np.where(diff >= 0, _S * (1.0 - _S) ** diff, 0.0)
    L = jnp.asarray(L, dtype=jnp.float32)

    return pl.pallas_call(
        _pcen_kernel,
        grid=(B // _BB,),
        in_specs=[
            pl.BlockSpec((_BB, F, T), lambda b: (b, 0, 0)),
            pl.BlockSpec((_TC, _TC), lambda b: (0, 0)),
        ],
        out_specs=pl.BlockSpec((_BB, F, T), lambda b: (b, 0, 0)),
        out_shape=jax.ShapeDtypeStruct((B, F, T), jnp.float32),
        compiler_params=pltpu.CompilerParams(
            dimension_semantics=("parallel",)
        ),
    )(x, L)
